# Initial kernel scaffold; baseline (speedup 1.0000x reference)
import jax
import jax.numpy as jnp
from jax.experimental import pallas as pl


def kernel(inputs, fixed_weight, variable_weight):
    raise NotImplementedError("write your pallas kernel here")


# speedup vs baseline: 1.3996x; 1.3996x over previous
import functools

import jax
import jax.numpy as jnp
from jax import lax
from jax.experimental import pallas as pl
from jax.experimental.pallas import tpu as pltpu
from jax.experimental.pallas import tpu_sc as plsc

NC = 2
NS = 16
L = 16
NW = NC * NS
G = 128


def _body(n_fixed, bpw, cap, base_args):
    (idx_hbm, fixed_hbm, var_hbm, out_hbm,
     idx_v, tidx_v, pos_v, rows_v, gsem, ssem) = base_args
    nch_cap = cap // G

    wid = lax.axis_index("s") * NC + lax.axis_index("c")
    base = wid * bpw
    pltpu.sync_copy(idx_hbm.at[pl.ds(base, bpw)], idx_v)
    iota = lax.iota(jnp.int32, L)

    def part_body(j, carry):
        cF, cV = carry
        v = idx_v[pl.ds(j * L, L)]
        m = v < n_fixed
        mi = m.astype(jnp.int32)
        dF = cF + plsc.cumsum(mi) - 1
        dV = (cap - 1) - (cV + plsc.cumsum(1 - mi) - 1)
        p = base + j * L + iota
        plsc.store_scatter(tidx_v, [dF >> 7, dF & (G - 1)], v, mask=m)
        plsc.store_scatter(pos_v, [dF >> 7, dF & (G - 1)], p, mask=m)
        plsc.store_scatter(tidx_v, [dV >> 7, dV & (G - 1)], v - n_fixed, mask=~m)
        plsc.store_scatter(pos_v, [dV >> 7, dV & (G - 1)], p, mask=~m)
        cnt = plsc.all_reduce_population_count(m)
        return cF + cnt, cV + (L - cnt)

    z = jnp.zeros((L,), jnp.int32)
    cF, cV = lax.fori_loop(0, bpw // L, part_body, (z, z))
    nF = jnp.max(cF)
    nV = jnp.max(cV)
    rF = ((nF + G - 1) // G) * G
    rV = ((nV + G - 1) // G) * G

    vF = tidx_v[0, pl.ds(0, L)]
    pF = pos_v[0, pl.ds(0, L)]
    eF_idx = jnp.sum(jnp.where(iota == 0, vF, 0))
    eF_pos = jnp.sum(jnp.where(iota == 0, pF, 0))
    vV = tidx_v[nch_cap - 1, pl.ds(G - L, L)]
    pV = pos_v[nch_cap - 1, pl.ds(G - L, L)]
    eV_idx = jnp.sum(jnp.where(iota == L - 1, vV, 0))
    eV_pos = jnp.sum(jnp.where(iota == L - 1, pV, 0))

    def pad_f(t, _):
        f = nF + t * L + iota
        mk = f < rF
        plsc.store_scatter(tidx_v, [f >> 7, f & (G - 1)],
                           jnp.full((L,), eF_idx, jnp.int32), mask=mk)
        plsc.store_scatter(pos_v, [f >> 7, f & (G - 1)],
                           jnp.full((L,), eF_pos, jnp.int32), mask=mk)
        return 0

    lax.fori_loop(0, (rF - nF + L - 1) // L, pad_f, 0)

    def pad_v(t, _):
        f = (cap - rV) + t * L + iota
        mk = f < cap - nV
        plsc.store_scatter(tidx_v, [f >> 7, f & (G - 1)],
                           jnp.full((L,), eV_idx, jnp.int32), mask=mk)
        plsc.store_scatter(pos_v, [f >> 7, f & (G - 1)],
                           jnp.full((L,), eV_pos, jnp.int32), mask=mk)
        return 0

    lax.fori_loop(0, (rV - nV + L - 1) // L, pad_v, 0)

    def gs_loop(c0, nch, table):
        def body(c_rel, _):
            c = c0 + c_rel
            pltpu.async_copy(table.at[tidx_v.at[c]], rows_v, gsem).wait()
            pltpu.async_copy(rows_v, out_hbm.at[pos_v.at[c]], ssem).wait()
            return 0
        lax.fori_loop(0, nch, body, 0)

    gs_loop(0, rF // G, fixed_hbm)
    gs_loop(nch_cap - rV // G, rV // G, var_hbm)


@functools.lru_cache(maxsize=None)
def _build(total, n_fixed, n_var, d):
    assert total % (NW * L) == 0
    bpw = total // NW
    cap = bpw + 2 * G
    nch_cap = cap // G

    mesh = plsc.VectorSubcoreMesh(core_axis_name="c", subcore_axis_name="s")

    @functools.partial(
        pl.kernel,
        out_type=jax.ShapeDtypeStruct((total, d), jnp.float32),
        mesh=mesh,
        scratch_types=[
            pltpu.VMEM((bpw,), jnp.int32),
            pltpu.VMEM((nch_cap, G), jnp.int32),
            pltpu.VMEM((nch_cap, G), jnp.int32),
            pltpu.VMEM((G, d), jnp.float32),
            pltpu.SemaphoreType.DMA,
            pltpu.SemaphoreType.DMA,
        ],
        compiler_params=pltpu.CompilerParams(
            needs_layout_passes=False, use_tc_tiling_on_sc=False
        ),
    )
    def k(*args):
        _body(n_fixed, bpw, cap, args)

    return k


def kernel(inputs, fixed_weight, variable_weight):
    b, s = inputs.shape
    n_fixed, d = fixed_weight.shape
    n_var = variable_weight.shape[0]
    idx = inputs.reshape(b * s).astype(jnp.int32)
    out = _build(b * s, n_fixed, n_var, d)(idx, fixed_weight, variable_weight)
    return out.reshape(b, s, d)


# speedup vs baseline: 1.5912x; 1.1369x over previous
import functools

import jax
import jax.numpy as jnp
from jax import lax
from jax.experimental import pallas as pl
from jax.experimental.pallas import tpu as pltpu
from jax.experimental.pallas import tpu_sc as plsc

NC = 2
NS = 16
L = 16
NW = NC * NS
G = 128
NB = 4


def _body(n_fixed, bpw, cap, base_args):
    (idx_hbm, fixed_hbm, var_hbm, out_hbm,
     idx_v, tidx_v, pos_v, rows_v, *sems) = base_args
    gsems, ssems = sems[:NB], sems[NB:]
    nch_cap = cap // G

    wid = lax.axis_index("s") * NC + lax.axis_index("c")
    base = wid * bpw
    pltpu.sync_copy(idx_hbm.at[pl.ds(base, bpw)], idx_v)
    iota = lax.iota(jnp.int32, L)

    def part_body(j, carry):
        cF, cV = carry
        v = idx_v[pl.ds(j * L, L)]
        m = v < n_fixed
        mi = m.astype(jnp.int32)
        dF = cF + plsc.cumsum(mi) - 1
        dV = (cap - 1) - (cV + plsc.cumsum(1 - mi) - 1)
        p = base + j * L + iota
        plsc.store_scatter(tidx_v, [dF >> 7, dF & (G - 1)], v, mask=m)
        plsc.store_scatter(pos_v, [dF >> 7, dF & (G - 1)], p, mask=m)
        plsc.store_scatter(tidx_v, [dV >> 7, dV & (G - 1)], v - n_fixed, mask=~m)
        plsc.store_scatter(pos_v, [dV >> 7, dV & (G - 1)], p, mask=~m)
        cnt = plsc.all_reduce_population_count(m)
        return cF + cnt, cV + (L - cnt)

    z = jnp.zeros((L,), jnp.int32)
    cF, cV = lax.fori_loop(0, bpw // L, part_body, (z, z))
    nF = jnp.max(cF)
    nV = jnp.max(cV)
    rF = ((nF + G - 1) // G) * G
    rV = ((nV + G - 1) // G) * G

    vF = tidx_v[0, pl.ds(0, L)]
    pF = pos_v[0, pl.ds(0, L)]
    eF_idx = jnp.sum(jnp.where(iota == 0, vF, 0))
    eF_pos = jnp.sum(jnp.where(iota == 0, pF, 0))
    vV = tidx_v[nch_cap - 1, pl.ds(G - L, L)]
    pV = pos_v[nch_cap - 1, pl.ds(G - L, L)]
    eV_idx = jnp.sum(jnp.where(iota == L - 1, vV, 0))
    eV_pos = jnp.sum(jnp.where(iota == L - 1, pV, 0))

    def pad_f(t, _):
        f = nF + t * L + iota
        mk = f < rF
        plsc.store_scatter(tidx_v, [f >> 7, f & (G - 1)],
                           jnp.full((L,), eF_idx, jnp.int32), mask=mk)
        plsc.store_scatter(pos_v, [f >> 7, f & (G - 1)],
                           jnp.full((L,), eF_pos, jnp.int32), mask=mk)
        return 0

    lax.fori_loop(0, (rF - nF + L - 1) // L, pad_f, 0)

    def pad_v(t, _):
        f = (cap - rV) + t * L + iota
        mk = f < cap - nV
        plsc.store_scatter(tidx_v, [f >> 7, f & (G - 1)],
                           jnp.full((L,), eV_idx, jnp.int32), mask=mk)
        plsc.store_scatter(pos_v, [f >> 7, f & (G - 1)],
                           jnp.full((L,), eV_pos, jnp.int32), mask=mk)
        return 0

    lax.fori_loop(0, (rV - nV + L - 1) // L, pad_v, 0)

    def gs_loop(c0, nch, table):
        ngrp = (nch + NB - 1) // NB

        def grp(g, _):
            for b in range(NB):
                i = g * NB + b
                c = c0 + i

                @pl.when(jnp.logical_and(i < nch, g > 0))
                def _():
                    pltpu.make_async_copy(
                        rows_v.at[b], out_hbm.at[pos_v.at[c - NB]], ssems[b]
                    ).wait()

                @pl.when(i < nch)
                def _():
                    pltpu.make_async_copy(
                        table.at[tidx_v.at[c]], rows_v.at[b], gsems[b]
                    ).start()

            for b in range(NB):
                i = g * NB + b
                c = c0 + i

                @pl.when(i < nch)
                def _():
                    pltpu.make_async_copy(
                        table.at[tidx_v.at[c]], rows_v.at[b], gsems[b]
                    ).wait()
                    pltpu.make_async_copy(
                        rows_v.at[b], out_hbm.at[pos_v.at[c]], ssems[b]
                    ).start()

            return 0

        lax.fori_loop(0, ngrp, grp, 0)
        for b in range(NB):
            i_b = (jnp.maximum(nch - 1 - b, 0) // NB) * NB + b

            @pl.when(b < nch)
            def _():
                pltpu.make_async_copy(
                    rows_v.at[b], out_hbm.at[pos_v.at[c0 + i_b]], ssems[b]
                ).wait()

    gs_loop(0, rF // G, fixed_hbm)
    gs_loop(nch_cap - rV // G, rV // G, var_hbm)


@functools.lru_cache(maxsize=None)
def _build(total, n_fixed, n_var, d):
    assert total % (NW * L) == 0
    bpw = total // NW
    cap = bpw + 2 * G
    nch_cap = cap // G

    mesh = plsc.VectorSubcoreMesh(core_axis_name="c", subcore_axis_name="s")

    @functools.partial(
        pl.kernel,
        out_type=jax.ShapeDtypeStruct((total, d), jnp.float32),
        mesh=mesh,
        scratch_types=[
            pltpu.VMEM((bpw,), jnp.int32),
            pltpu.VMEM((nch_cap, G), jnp.int32),
            pltpu.VMEM((nch_cap, G), jnp.int32),
            pltpu.VMEM((NB, G, d), jnp.float32),
        ] + [pltpu.SemaphoreType.DMA] * (2 * NB),
        compiler_params=pltpu.CompilerParams(
            needs_layout_passes=False, use_tc_tiling_on_sc=False
        ),
    )
    def k(*args):
        _body(n_fixed, bpw, cap, args)

    return k


def kernel(inputs, fixed_weight, variable_weight):
    b, s = inputs.shape
    n_fixed, d = fixed_weight.shape
    n_var = variable_weight.shape[0]
    idx = inputs.reshape(b * s).astype(jnp.int32)
    out = _build(b * s, n_fixed, n_var, d)(idx, fixed_weight, variable_weight)
    return out.reshape(b, s, d)


# speedup vs baseline: 1.6696x; 1.0493x over previous
import functools

import jax
import jax.numpy as jnp
from jax import lax
from jax.experimental import pallas as pl
from jax.experimental.pallas import tpu as pltpu
from jax.experimental.pallas import tpu_sc as plsc

NC = 2
NS = 16
L = 16
NW = NC * NS
G = 128
NB = 4


def _body(n_fixed, bpw, cap, s_len, b_len, base_args):
    (idx_hbm, fixed_hbm, var_hbm, out_hbm,
     idx_v, tidx_v, pos_v, rows_v, *sems) = base_args
    gsems, ssems = sems[:NB], sems[NB:]
    nch_cap = cap // G

    wid = lax.axis_index("s") * NC + lax.axis_index("c")
    base = wid * bpw
    pltpu.sync_copy(idx_hbm.at[pl.ds(base, bpw)], idx_v)
    iota = lax.iota(jnp.int32, L)

    def part_body(j, carry):
        cF, cV, s0, b0 = carry
        v = idx_v[pl.ds(j * L, L)]
        m = v < n_fixed
        mi = m.astype(jnp.int32)
        dF = cF + plsc.cumsum(mi) - 1
        dV = (cap - 1) - (cV + plsc.cumsum(1 - mi) - 1)
        s_v = s0 + iota
        wrap = (s_v >= s_len).astype(jnp.int32)
        p = (s_v - s_len * wrap) * b_len + (b0 + wrap)
        plsc.store_scatter(tidx_v, [dF >> 7, dF & (G - 1)], v, mask=m)
        plsc.store_scatter(pos_v, [dF >> 7, dF & (G - 1)], p, mask=m)
        plsc.store_scatter(tidx_v, [dV >> 7, dV & (G - 1)], v - n_fixed, mask=~m)
        plsc.store_scatter(pos_v, [dV >> 7, dV & (G - 1)], p, mask=~m)
        cnt = plsc.all_reduce_population_count(m)
        s0n = s0 + L
        w = (s0n >= s_len).astype(jnp.int32)
        return cF + cnt, cV + (L - cnt), s0n - s_len * w, b0 + w

    z = jnp.zeros((L,), jnp.int32)
    cF, cV, _, _ = lax.fori_loop(
        0, bpw // L, part_body,
        (z, z, jnp.int32(0), wid * (bpw // s_len)))
    nF = jnp.max(cF)
    nV = jnp.max(cV)
    rF = ((nF + G - 1) // G) * G
    rV = ((nV + G - 1) // G) * G

    vF = tidx_v[0, pl.ds(0, L)]
    pF = pos_v[0, pl.ds(0, L)]
    eF_idx = jnp.sum(jnp.where(iota == 0, vF, 0))
    eF_pos = jnp.sum(jnp.where(iota == 0, pF, 0))
    vV = tidx_v[nch_cap - 1, pl.ds(G - L, L)]
    pV = pos_v[nch_cap - 1, pl.ds(G - L, L)]
    eV_idx = jnp.sum(jnp.where(iota == L - 1, vV, 0))
    eV_pos = jnp.sum(jnp.where(iota == L - 1, pV, 0))

    def pad_f(t, _):
        f = nF + t * L + iota
        mk = f < rF
        plsc.store_scatter(tidx_v, [f >> 7, f & (G - 1)],
                           jnp.full((L,), eF_idx, jnp.int32), mask=mk)
        plsc.store_scatter(pos_v, [f >> 7, f & (G - 1)],
                           jnp.full((L,), eF_pos, jnp.int32), mask=mk)
        return 0

    lax.fori_loop(0, (rF - nF + L - 1) // L, pad_f, 0)

    def pad_v(t, _):
        f = (cap - rV) + t * L + iota
        mk = f < cap - nV
        plsc.store_scatter(tidx_v, [f >> 7, f & (G - 1)],
                           jnp.full((L,), eV_idx, jnp.int32), mask=mk)
        plsc.store_scatter(pos_v, [f >> 7, f & (G - 1)],
                           jnp.full((L,), eV_pos, jnp.int32), mask=mk)
        return 0

    lax.fori_loop(0, (rV - nV + L - 1) // L, pad_v, 0)

    def gs_loop(c0, nch, table):
        ngrp = (nch + NB - 1) // NB

        def grp(g, _):
            for b in range(NB):
                i = g * NB + b
                c = c0 + i

                @pl.when(jnp.logical_and(i < nch, g > 0))
                def _():
                    pltpu.make_async_copy(
                        rows_v.at[b], out_hbm.at[pos_v.at[c - NB]], ssems[b]
                    ).wait()

                @pl.when(i < nch)
                def _():
                    pltpu.make_async_copy(
                        table.at[tidx_v.at[c]], rows_v.at[b], gsems[b]
                    ).start()

            for b in range(NB):
                i = g * NB + b
                c = c0 + i

                @pl.when(i < nch)
                def _():
                    pltpu.make_async_copy(
                        table.at[tidx_v.at[c]], rows_v.at[b], gsems[b]
                    ).wait()
                    pltpu.make_async_copy(
                        rows_v.at[b], out_hbm.at[pos_v.at[c]], ssems[b]
                    ).start()

            return 0

        lax.fori_loop(0, ngrp, grp, 0)
        for b in range(NB):
            i_b = (jnp.maximum(nch - 1 - b, 0) // NB) * NB + b

            @pl.when(b < nch)
            def _():
                pltpu.make_async_copy(
                    rows_v.at[b], out_hbm.at[pos_v.at[c0 + i_b]], ssems[b]
                ).wait()

    gs_loop(0, rF // G, fixed_hbm)
    gs_loop(nch_cap - rV // G, rV // G, var_hbm)


@functools.lru_cache(maxsize=None)
def _build(s_len, b_len, n_fixed, n_var, d):
    total = s_len * b_len
    assert total % (NW * L) == 0
    bpw = total // NW
    assert bpw % s_len == 0
    cap = bpw + 2 * G
    nch_cap = cap // G

    mesh = plsc.VectorSubcoreMesh(core_axis_name="c", subcore_axis_name="s")

    @functools.partial(
        pl.kernel,
        out_type=jax.ShapeDtypeStruct((total, d), jnp.float32),
        mesh=mesh,
        scratch_types=[
            pltpu.VMEM((bpw,), jnp.int32),
            pltpu.VMEM((nch_cap, G), jnp.int32),
            pltpu.VMEM((nch_cap, G), jnp.int32),
            pltpu.VMEM((NB, G, d), jnp.float32),
        ] + [pltpu.SemaphoreType.DMA] * (2 * NB),
        compiler_params=pltpu.CompilerParams(
            needs_layout_passes=False, use_tc_tiling_on_sc=False
        ),
    )
    def k(*args):
        _body(n_fixed, bpw, cap, s_len, b_len, args)

    return k


def kernel(inputs, fixed_weight, variable_weight):
    b, s = inputs.shape
    n_fixed, d = fixed_weight.shape
    n_var = variable_weight.shape[0]
    idx = inputs.reshape(b * s).astype(jnp.int32)
    w128 = n_fixed * d // 128
    fw = lax.optimization_barrier(
        fixed_weight.reshape(w128, 128)).reshape(n_fixed, d)
    vw = lax.optimization_barrier(
        variable_weight.reshape(n_var * d // 128, 128)).reshape(n_var, d)
    out = _build(s, b, n_fixed, n_var, d)(idx, fw, vw)
    return jnp.transpose(out.reshape(s, b, d), (1, 0, 2))
